# baseline (speedup 1.0000x reference)
import functools

import jax
import jax.numpy as jnp
from jax import lax
from jax.experimental import pallas as pl
from jax.experimental.pallas import tpu as pltpu
from jax.experimental.pallas import tpu_sc as plsc

N = 10000
E = 160000
D = 256
H = 256
DE = 16
MLP_H = 96

NP = 10240
MH = 128
NC = 2
NS = 16
NW = NC * NS
HALF = D // 2
L = 16

EPT_A = E // NS
CHUNK_A = 80
NCH_A = EPT_A // CHUNK_A
ROWS_PT = NP // NS

EPT_C = E // NW
CHUNK_C = 128
NFULL_C = EPT_C // CHUNK_C
TAIL_C = EPT_C - NFULL_C * CHUNK_C
NCH_C = NFULL_C + 1

RB = 2048
EB = 8000

_SC_MESH = plsc.VectorSubcoreMesh(core_axis_name="c", subcore_axis_name="s")
_SC_PARAMS = pltpu.CompilerParams(needs_layout_passes=False)


def _seg_sum_body(with_cnt, x2, src_off, dst, z128,
                  *refs):
    if with_cnt:
        (sum_out, cnt_out, is0, id0, is1, id1, rb0, rb1, hist, acc_sp,
         sg0, sg1) = refs
    else:
        sum_out, is0, id0, is1, id1, rb0, rb1, acc_sp, sg0, sg1 = refs
    c = lax.axis_index("c")
    s = lax.axis_index("s")
    row0 = s * ROWS_PT
    pltpu.sync_copy(z128, acc_sp.at[pl.ds(row0, ROWS_PT)])
    if with_cnt:
        zv = jnp.zeros((L,), jnp.float32)

        def zrow(k, cc):
            hist[pl.ds(k * L, L)] = zv
            return cc

        lax.fori_loop(0, NP // L, zrow, 0)
    plsc.subcore_barrier()
    base_s = c * E + s * EPT_A
    base_d = s * EPT_A
    ones_v = jnp.full((L,), 1.0, jnp.float32)

    def load_idx(j, is_b, id_b):
        pltpu.sync_copy(src_off.at[pl.ds(base_s + j * CHUNK_A, CHUNK_A)],
                        is_b)
        pltpu.sync_copy(dst.at[pl.ds(base_d + j * CHUNK_A, CHUNK_A)], id_b)

    def issue_gather(is_b, rb, sg):
        pltpu.async_copy(x2.at[is_b], rb, sg)

    def wait_gather(rb, sg):
        pltpu.make_async_copy(x2.at[pl.ds(0, CHUNK_A)], rb, sg).wait()

    def consume(rb, id_b):
        pltpu.sync_copy(rb, acc_sp.at[id_b], add=True)
        if with_cnt:
            for t in range(CHUNK_A // L):
                idx16 = id_b[pl.ds(t * L, L)]
                plsc.addupdate_scatter(hist, [idx16], ones_v)

    load_idx(0, is0, id0)
    issue_gather(is0, rb0, sg0)
    load_idx(1, is1, id1)
    npair = (NCH_A - 1) // 2

    def pipe(g, carry):
        j0 = 2 * g
        issue_gather(is1, rb1, sg1)
        wait_gather(rb0, sg0)
        consume(rb0, id0)
        load_idx(j0 + 2, is0, id0)
        issue_gather(is0, rb0, sg0)
        wait_gather(rb1, sg1)
        consume(rb1, id1)

        @pl.when(g < npair - 1)
        def _():
            load_idx(j0 + 3, is1, id1)

        return carry

    lax.fori_loop(0, npair, pipe, 0)
    wait_gather(rb0, sg0)
    consume(rb0, id0)
    plsc.subcore_barrier()
    pltpu.sync_copy(acc_sp.at[pl.ds(row0, ROWS_PT)],
                    sum_out.at[pl.ds(c * NP + row0, ROWS_PT)])
    if with_cnt:
        w = c * NS + s
        pltpu.sync_copy(hist, cnt_out.at[pl.ds(w * NP, NP)])


def _seg_sum_call(x2, src_idx, dst_idx, z128, with_cnt):
    out_type = [jax.ShapeDtypeStruct((2 * NP, HALF), jnp.float32)]
    scratch = [
        pltpu.VMEM((CHUNK_A,), jnp.int32),
        pltpu.VMEM((CHUNK_A,), jnp.int32),
        pltpu.VMEM((CHUNK_A,), jnp.int32),
        pltpu.VMEM((CHUNK_A,), jnp.int32),
        pltpu.VMEM((CHUNK_A, HALF), jnp.float32),
        pltpu.VMEM((CHUNK_A, HALF), jnp.float32),
    ]
    if with_cnt:
        out_type.append(jax.ShapeDtypeStruct((NW * NP,), jnp.float32))
        scratch.append(pltpu.VMEM((NP,), jnp.float32))
    scratch.append(pltpu.VMEM_SHARED((NP, HALF), jnp.float32))
    scratch.append(pltpu.SemaphoreType.DMA)
    scratch.append(pltpu.SemaphoreType.DMA)
    fn = pl.kernel(
        functools.partial(_seg_sum_body, with_cnt),
        out_type=tuple(out_type) if with_cnt else out_type[0],
        mesh=_SC_MESH,
        scratch_types=tuple(scratch),
        compiler_params=_SC_PARAMS,
        name="sc_seg_sum" + ("_cnt" if with_cnt else ""),
    )
    return fn(x2, src_idx, dst_idx, z128)


def _edge_gather_body(pq, uv_idx, gu_out, gv_out,
                      idx_u, idx_v, ru0, rv0, ru1, rv1, sg0, sg1):
    c = lax.axis_index("c")
    s = lax.axis_index("s")
    w = c * NS + s
    pltpu.sync_copy(uv_idx.at[w], idx_u)
    pltpu.sync_copy(uv_idx.at[w + NW], idx_v)

    def issue(j, ru_b, rv_b, sg):
        pltpu.async_copy(pq.at[idx_u.at[j]], ru_b, sg)
        pltpu.async_copy(pq.at[idx_v.at[j]], rv_b, sg)

    def wait2(ru_b, rv_b, sg):
        pltpu.make_async_copy(pq.at[pl.ds(0, CHUNK_C)], ru_b, sg).wait()
        pltpu.make_async_copy(pq.at[pl.ds(0, CHUNK_C)], rv_b, sg).wait()

    def store(j, ru_b, rv_b):
        sl = pl.ds(w * EPT_C + j * CHUNK_C, CHUNK_C)
        pltpu.sync_copy(ru_b, gu_out.at[sl])
        pltpu.sync_copy(rv_b, gv_out.at[sl])

    issue(0, ru0, rv0, sg0)
    npair = NFULL_C // 2

    def pipe(g, carry):
        j0 = 2 * g
        issue(j0 + 1, ru1, rv1, sg1)
        wait2(ru0, rv0, sg0)
        store(j0, ru0, rv0)
        issue(j0 + 2, ru0, rv0, sg0)
        wait2(ru1, rv1, sg1)
        store(j0 + 1, ru1, rv1)
        return carry

    lax.fori_loop(0, npair, pipe, 0)
    issue(NFULL_C, ru1, rv1, sg1)
    wait2(ru0, rv0, sg0)
    store(NFULL_C - 1, ru0, rv0)
    wait2(ru1, rv1, sg1)
    sl_t = pl.ds(w * EPT_C + NFULL_C * CHUNK_C, TAIL_C)
    pltpu.sync_copy(ru1.at[pl.ds(0, TAIL_C)], gu_out.at[sl_t])
    pltpu.sync_copy(rv1.at[pl.ds(0, TAIL_C)], gv_out.at[sl_t])


_edge_gather = pl.kernel(
    _edge_gather_body,
    out_type=(jax.ShapeDtypeStruct((E, MH), jnp.float32),
              jax.ShapeDtypeStruct((E, MH), jnp.float32)),
    mesh=_SC_MESH,
    scratch_types=(
        pltpu.VMEM((NCH_C, CHUNK_C), jnp.int32),
        pltpu.VMEM((NCH_C, CHUNK_C), jnp.int32),
        pltpu.VMEM((CHUNK_C, MH), jnp.float32),
        pltpu.VMEM((CHUNK_C, MH), jnp.float32),
        pltpu.VMEM((CHUNK_C, MH), jnp.float32),
        pltpu.VMEM((CHUNK_C, MH), jnp.float32),
        pltpu.SemaphoreType.DMA,
        pltpu.SemaphoreType.DMA,
    ),
    compiler_params=_SC_PARAMS,
    name="sc_edge_gather",
)


def _inv_degree(cnt_block):
    deg2 = jnp.sum(jnp.transpose(cnt_block), axis=1, keepdims=True)
    return 1.0 / jnp.maximum(0.5 * deg2, 1.0)


def _tc_layer1_body(sumA, sumB, cnt, x, wl, wr, b, out):
    inv = _inv_degree(cnt[...])
    mean = jnp.concatenate([sumA[...], sumB[...]], axis=1) * inv
    acc = jnp.dot(mean, wl[...], preferred_element_type=jnp.float32)
    acc += jnp.dot(x[...], wr[...], preferred_element_type=jnp.float32)
    out[...] = jnp.maximum(acc + b[...], 0.0)


def _tc_layer1(sum1, cnt, xp, W_l1, W_r1, b_l1):
    nb = NP // RB
    return pl.pallas_call(
        _tc_layer1_body,
        grid=(nb, 2),
        in_specs=[
            pl.BlockSpec((RB, HALF), lambda i, h: (i, 0)),
            pl.BlockSpec((RB, HALF), lambda i, h: (i + nb, 0)),
            pl.BlockSpec((NW, RB), lambda i, h: (0, i)),
            pl.BlockSpec((RB, D), lambda i, h: (i, 0)),
            pl.BlockSpec((D, HALF), lambda i, h: (0, h)),
            pl.BlockSpec((D, HALF), lambda i, h: (0, h)),
            pl.BlockSpec((1, HALF), lambda i, h: (0, h)),
        ],
        out_specs=pl.BlockSpec((RB, HALF), lambda i, h: (h * nb + i, 0)),
        out_shape=jax.ShapeDtypeStruct((2 * NP, HALF), jnp.float32),
        name="tc_layer1",
    )(sum1, sum1, cnt, xp, W_l1, W_r1, b_l1.reshape(1, H))


def _tc_layer2_body(sumA, sumB, cnt, z1A, z1B, wl, wr, b, wu, wv,
                    pu_out, pv_out):
    h = pl.program_id(1)
    inv = _inv_degree(cnt[...])
    mean = jnp.concatenate([sumA[...], sumB[...]], axis=1) * inv
    z1 = jnp.concatenate([z1A[...], z1B[...]], axis=1)
    z2h = (jnp.dot(mean, wl[...], preferred_element_type=jnp.float32)
           + jnp.dot(z1, wr[...], preferred_element_type=jnp.float32)
           + b[...])
    pu_part = jnp.dot(z2h, wu[...], preferred_element_type=jnp.float32)
    pv_part = jnp.dot(z2h, wv[...], preferred_element_type=jnp.float32)

    @pl.when(h == 0)
    def _():
        pu_out[...] = pu_part
        pv_out[...] = pv_part

    @pl.when(h != 0)
    def _():
        pu_out[...] += pu_part
        pv_out[...] += pv_part


def _tc_layer2(sum2, cnt, z1s, W_l2, W_r2, b_l2, Wu, Wv):
    nb = NP // RB
    return pl.pallas_call(
        _tc_layer2_body,
        grid=(nb, 2),
        in_specs=[
            pl.BlockSpec((RB, HALF), lambda i, h: (i, 0)),
            pl.BlockSpec((RB, HALF), lambda i, h: (i + nb, 0)),
            pl.BlockSpec((NW, RB), lambda i, h: (0, i)),
            pl.BlockSpec((RB, HALF), lambda i, h: (i, 0)),
            pl.BlockSpec((RB, HALF), lambda i, h: (i + nb, 0)),
            pl.BlockSpec((H, HALF), lambda i, h: (0, h)),
            pl.BlockSpec((H, HALF), lambda i, h: (0, h)),
            pl.BlockSpec((1, HALF), lambda i, h: (0, h)),
            pl.BlockSpec((HALF, MH), lambda i, h: (h, 0)),
            pl.BlockSpec((HALF, MH), lambda i, h: (h, 0)),
        ],
        out_specs=[
            pl.BlockSpec((RB, MH), lambda i, h: (i, 0)),
            pl.BlockSpec((RB, MH), lambda i, h: (i, 0)),
        ],
        out_shape=[
            jax.ShapeDtypeStruct((NP, MH), jnp.float32),
            jax.ShapeDtypeStruct((NP, MH), jnp.float32),
        ],
        name="tc_layer2",
    )(sum2, sum2, cnt, z1s, z1s, W_l2, W_r2, b_l2.reshape(1, H), Wu, Wv)


def _tc_head_body(gu, gv, ea, we, bm1, wm2, bm2, out):
    pre = (gu[...] + gv[...] + jnp.dot(ea[...], we[...],
                                       preferred_element_type=jnp.float32)
           + bm1[...])
    hr = jnp.maximum(pre, 0.0)
    logit = jnp.dot(hr, wm2[...], preferred_element_type=jnp.float32) + bm2[...]
    out[...] = 1.0 / (1.0 + jnp.exp(-logit))


def _tc_head(gu, gv, edge_attr, We, b_m1, W_m2, b_m2):
    return pl.pallas_call(
        _tc_head_body,
        grid=(E // EB,),
        in_specs=[
            pl.BlockSpec((EB, MH), lambda i: (i, 0)),
            pl.BlockSpec((EB, MH), lambda i: (i, 0)),
            pl.BlockSpec((EB, DE), lambda i: (i, 0)),
            pl.BlockSpec((DE, MH), lambda i: (0, 0)),
            pl.BlockSpec((1, MH), lambda i: (0, 0)),
            pl.BlockSpec((MH, 1), lambda i: (0, 0)),
            pl.BlockSpec((1, 1), lambda i: (0, 0)),
        ],
        out_specs=pl.BlockSpec((EB, 1), lambda i: (i, 0)),
        out_shape=jax.ShapeDtypeStruct((E, 1), jnp.float32),
        name="tc_head",
    )(gu, gv, edge_attr, We, b_m1, W_m2, b_m2.reshape(1, 1))


def _pad_mh(w):
    return jnp.pad(w, ((0, 0), (0, MH - MLP_H)))


def kernel(x, edge_index, edge_u, edge_v, edge_attr,
           W_l1, b_l1, W_r1, W_l2, b_l2, W_r2,
           W_m1, b_m1, W_m2, b_m2):
    src = edge_index[0].astype(jnp.int32)
    dst = edge_index[1].astype(jnp.int32)

    src_idx = jnp.concatenate([src, src + NP])
    dst_idx = dst
    z128 = jnp.zeros((ROWS_PT, HALF), jnp.float32)

    xp = jnp.pad(x, ((0, NP - N), (0, 0)))
    x2 = jnp.concatenate([xp[:, :HALF], xp[:, HALF:]], axis=0)
    sum1, cnt = _seg_sum_call(x2, src_idx, dst_idx, z128, True)
    cnt = cnt.reshape(NW, NP)
    z1s = _tc_layer1(sum1, cnt, xp, W_l1, W_r1, b_l1)

    sum2 = _seg_sum_call(z1s, src_idx, dst_idx, z128, False)
    Wu = _pad_mh(W_m1[:H])
    Wv = _pad_mh(W_m1[H:2 * H])
    We = _pad_mh(W_m1[2 * H:])
    bm1 = _pad_mh(b_m1.reshape(1, MLP_H))
    wm2 = jnp.pad(W_m2, ((0, MH - MLP_H), (0, 0)))
    pu, pv = _tc_layer2(sum2, cnt, z1s, W_l2, W_r2, b_l2, Wu, Wv)

    pq = jnp.concatenate([pu, pv], axis=0)
    pad_c = NCH_C * CHUNK_C - EPT_C
    u_r = jnp.pad(edge_u.astype(jnp.int32).reshape(NW, EPT_C),
                  ((0, 0), (0, pad_c))).reshape(NW, NCH_C, CHUNK_C)
    v_r = jnp.pad(edge_v.astype(jnp.int32).reshape(NW, EPT_C),
                  ((0, 0), (0, pad_c))).reshape(NW, NCH_C, CHUNK_C) + NP
    uv_idx = jnp.concatenate([u_r, v_r], axis=0)
    gu, gv = _edge_gather(pq, uv_idx)
    out = _tc_head(gu, gv, edge_attr, We, bm1, wm2, b_m2)
    return out.reshape(E)


# speedup vs baseline: 1.2569x; 1.2569x over previous
import functools

import jax
import jax.numpy as jnp
from jax import lax
from jax.experimental import pallas as pl
from jax.experimental.pallas import tpu as pltpu
from jax.experimental.pallas import tpu_sc as plsc

N = 10000
E = 160000
D = 256
H = 256
DE = 16
MLP_H = 96

NP = 10240
MH = 128
NC = 2
NS = 16
NW = NC * NS
HALF = D // 2
L = 16

EPT_A = E // NS
CHUNK_A = 80
NCH_A = EPT_A // CHUNK_A
ROWS_PT = NP // NS

EPT_C = E // NW
CHUNK_C = 128
NFULL_C = EPT_C // CHUNK_C
TAIL_C = EPT_C - NFULL_C * CHUNK_C
NCH_C = NFULL_C + 1

RB = 2048
EB = 8000

_SC_MESH = plsc.VectorSubcoreMesh(core_axis_name="c", subcore_axis_name="s")
_SC_PARAMS = pltpu.CompilerParams(needs_layout_passes=False)


def _seg_sum_body(with_cnt, x2, src_off, dst, z128,
                  *refs):
    if with_cnt:
        (sum_out, cnt_out, is0, id0, is1, id1, rb0, rb1, hist, acc_sp,
         sg0, sg1, si0, si1) = refs
    else:
        (sum_out, is0, id0, is1, id1, rb0, rb1, acc_sp,
         sg0, sg1, si0, si1) = refs
    c = lax.axis_index("c")
    s = lax.axis_index("s")
    row0 = s * ROWS_PT
    pltpu.sync_copy(z128, acc_sp.at[pl.ds(row0, ROWS_PT)])
    if with_cnt:
        zv = jnp.zeros((L,), jnp.float32)

        def zrow(k, cc):
            hist[pl.ds(k * L, L)] = zv
            return cc

        lax.fori_loop(0, NP // L, zrow, 0)
    plsc.subcore_barrier()
    base_s = c * E + s * EPT_A
    base_d = s * EPT_A
    ones_v = jnp.full((L,), 1.0, jnp.float32)

    def issue_idx(j, is_b, id_b, si):
        pltpu.async_copy(src_off.at[pl.ds(base_s + j * CHUNK_A, CHUNK_A)],
                         is_b, si)
        pltpu.async_copy(dst.at[pl.ds(base_d + j * CHUNK_A, CHUNK_A)],
                         id_b, si)

    def wait_idx(is_b, id_b, si):
        pltpu.make_async_copy(src_off.at[pl.ds(0, CHUNK_A)], is_b, si).wait()
        pltpu.make_async_copy(dst.at[pl.ds(0, CHUNK_A)], id_b, si).wait()

    def issue_gather(is_b, rb, sg):
        pltpu.async_copy(x2.at[is_b], rb, sg)

    def wait_gather(rb, sg):
        pltpu.make_async_copy(x2.at[pl.ds(0, CHUNK_A)], rb, sg).wait()

    def consume(rb, id_b):
        pltpu.sync_copy(rb, acc_sp.at[id_b], add=True)
        if with_cnt:
            for t in range(CHUNK_A // L):
                idx16 = id_b[pl.ds(t * L, L)]
                plsc.addupdate_scatter(hist, [idx16], ones_v)

    issue_idx(0, is0, id0, si0)
    issue_idx(1, is1, id1, si1)
    wait_idx(is0, id0, si0)
    issue_gather(is0, rb0, sg0)
    npair = (NCH_A - 1) // 2

    def pipe(g, carry):
        j0 = 2 * g
        wait_idx(is1, id1, si1)
        issue_gather(is1, rb1, sg1)
        wait_gather(rb0, sg0)
        consume(rb0, id0)
        issue_idx(j0 + 2, is0, id0, si0)
        wait_gather(rb1, sg1)
        consume(rb1, id1)

        @pl.when(g < npair - 1)
        def _():
            issue_idx(j0 + 3, is1, id1, si1)

        wait_idx(is0, id0, si0)
        issue_gather(is0, rb0, sg0)
        return carry

    lax.fori_loop(0, npair, pipe, 0)
    wait_gather(rb0, sg0)
    consume(rb0, id0)
    plsc.subcore_barrier()
    pltpu.sync_copy(acc_sp.at[pl.ds(row0, ROWS_PT)],
                    sum_out.at[pl.ds(c * NP + row0, ROWS_PT)])
    if with_cnt:
        w = c * NS + s
        pltpu.sync_copy(hist, cnt_out.at[pl.ds(w * NP, NP)])


def _seg_sum_call(x2, src_idx, dst_idx, z128, with_cnt):
    out_type = [jax.ShapeDtypeStruct((2 * NP, HALF), jnp.float32)]
    scratch = [
        pltpu.VMEM((CHUNK_A,), jnp.int32),
        pltpu.VMEM((CHUNK_A,), jnp.int32),
        pltpu.VMEM((CHUNK_A,), jnp.int32),
        pltpu.VMEM((CHUNK_A,), jnp.int32),
        pltpu.VMEM((CHUNK_A, HALF), jnp.float32),
        pltpu.VMEM((CHUNK_A, HALF), jnp.float32),
    ]
    if with_cnt:
        out_type.append(jax.ShapeDtypeStruct((NW * NP,), jnp.float32))
        scratch.append(pltpu.VMEM((NP,), jnp.float32))
    scratch.append(pltpu.VMEM_SHARED((NP, HALF), jnp.float32))
    scratch.extend([pltpu.SemaphoreType.DMA] * 4)
    fn = pl.kernel(
        functools.partial(_seg_sum_body, with_cnt),
        out_type=tuple(out_type) if with_cnt else out_type[0],
        mesh=_SC_MESH,
        scratch_types=tuple(scratch),
        compiler_params=_SC_PARAMS,
        name="sc_seg_sum" + ("_cnt" if with_cnt else ""),
    )
    return fn(x2, src_idx, dst_idx, z128)


def _edge_gather_body(pq, uv_idx, out,
                      idx_u, idx_v, ru0, rv0, ru1, rv1,
                      sg0, sg1, ss0, ss1):
    c = lax.axis_index("c")
    s = lax.axis_index("s")
    w = c * NS + s
    pltpu.sync_copy(uv_idx.at[w], idx_u)
    pltpu.sync_copy(uv_idx.at[w + NW], idx_v)

    def issue(j, ru_b, rv_b, sg):
        pltpu.async_copy(pq.at[idx_u.at[j]], ru_b, sg)
        pltpu.async_copy(pq.at[idx_v.at[j]], rv_b, sg)

    def wait2(ru_b, rv_b, sg):
        pltpu.make_async_copy(pq.at[pl.ds(0, CHUNK_C)], ru_b, sg).wait()
        pltpu.make_async_copy(pq.at[pl.ds(0, CHUNK_C)], rv_b, sg).wait()

    def addrows(ru_b, rv_b, n):
        def addrow(k, cc):
            for r in range(2):
                for t in range(MH // L):
                    sl = pl.ds(t * L, L)
                    ru_b[2 * k + r, sl] = ru_b[2 * k + r, sl] + rv_b[2 * k + r, sl]
            return cc

        lax.fori_loop(0, n // 2, addrow, 0)

    def store_async(j, ru_b, ss):
        pltpu.async_copy(
            ru_b, out.at[pl.ds(w * EPT_C + j * CHUNK_C, CHUNK_C)], ss)

    def wait_store(ru_b, ss):
        pltpu.make_async_copy(pq.at[pl.ds(0, CHUNK_C)], ru_b, ss).wait()

    issue(0, ru0, rv0, sg0)
    npair = NFULL_C // 2

    def pipe(g, carry):
        j0 = 2 * g
        issue(j0 + 1, ru1, rv1, sg1)
        wait2(ru0, rv0, sg0)
        addrows(ru0, rv0, CHUNK_C)
        store_async(j0, ru0, ss0)
        wait2(ru1, rv1, sg1)
        addrows(ru1, rv1, CHUNK_C)
        wait_store(ru0, ss0)
        issue(j0 + 2, ru0, rv0, sg0)
        store_async(j0 + 1, ru1, ss1)
        wait_store(ru1, ss1)
        return carry

    lax.fori_loop(0, npair, pipe, 0)
    issue(NFULL_C, ru1, rv1, sg1)
    wait2(ru0, rv0, sg0)
    addrows(ru0, rv0, CHUNK_C)
    store_async(NFULL_C - 1, ru0, ss0)
    wait2(ru1, rv1, sg1)
    addrows(ru1, rv1, TAIL_C)
    wait_store(ru0, ss0)
    pltpu.sync_copy(ru1.at[pl.ds(0, TAIL_C)],
                    out.at[pl.ds(w * EPT_C + NFULL_C * CHUNK_C, TAIL_C)])


_edge_gather = pl.kernel(
    _edge_gather_body,
    out_type=jax.ShapeDtypeStruct((E, MH), jnp.float32),
    mesh=_SC_MESH,
    scratch_types=(
        pltpu.VMEM((NCH_C, CHUNK_C), jnp.int32),
        pltpu.VMEM((NCH_C, CHUNK_C), jnp.int32),
        pltpu.VMEM((CHUNK_C, MH), jnp.float32),
        pltpu.VMEM((CHUNK_C, MH), jnp.float32),
        pltpu.VMEM((CHUNK_C, MH), jnp.float32),
        pltpu.VMEM((CHUNK_C, MH), jnp.float32),
        pltpu.SemaphoreType.DMA,
        pltpu.SemaphoreType.DMA,
        pltpu.SemaphoreType.DMA,
        pltpu.SemaphoreType.DMA,
    ),
    compiler_params=_SC_PARAMS,
    name="sc_edge_gather",
)


def _inv_degree(cnt_block):
    deg2 = jnp.sum(jnp.transpose(cnt_block), axis=1, keepdims=True)
    return 1.0 / jnp.maximum(0.5 * deg2, 1.0)


def _tc_layer1_body(sumA, sumB, cnt, x, wl, wr, b, out):
    inv = _inv_degree(cnt[...])
    mean = jnp.concatenate([sumA[...], sumB[...]], axis=1) * inv
    acc = jnp.dot(mean, wl[...], preferred_element_type=jnp.float32)
    acc += jnp.dot(x[...], wr[...], preferred_element_type=jnp.float32)
    out[...] = jnp.maximum(acc + b[...], 0.0)


def _tc_layer1(sum1, cnt, xp, W_l1, W_r1, b_l1):
    nb = NP // RB
    return pl.pallas_call(
        _tc_layer1_body,
        grid=(nb, 2),
        in_specs=[
            pl.BlockSpec((RB, HALF), lambda i, h: (i, 0)),
            pl.BlockSpec((RB, HALF), lambda i, h: (i + nb, 0)),
            pl.BlockSpec((NW, RB), lambda i, h: (0, i)),
            pl.BlockSpec((RB, D), lambda i, h: (i, 0)),
            pl.BlockSpec((D, HALF), lambda i, h: (0, h)),
            pl.BlockSpec((D, HALF), lambda i, h: (0, h)),
            pl.BlockSpec((1, HALF), lambda i, h: (0, h)),
        ],
        out_specs=pl.BlockSpec((RB, HALF), lambda i, h: (h * nb + i, 0)),
        out_shape=jax.ShapeDtypeStruct((2 * NP, HALF), jnp.float32),
        name="tc_layer1",
    )(sum1, sum1, cnt, xp, W_l1, W_r1, b_l1.reshape(1, H))


def _tc_layer2_body(sumA, sumB, cnt, z1A, z1B, wl, wr, b, wu, wv,
                    pu_out, pv_out):
    h = pl.program_id(1)
    inv = _inv_degree(cnt[...])
    mean = jnp.concatenate([sumA[...], sumB[...]], axis=1) * inv
    z1 = jnp.concatenate([z1A[...], z1B[...]], axis=1)
    z2h = (jnp.dot(mean, wl[...], preferred_element_type=jnp.float32)
           + jnp.dot(z1, wr[...], preferred_element_type=jnp.float32)
           + b[...])
    pu_part = jnp.dot(z2h, wu[...], preferred_element_type=jnp.float32)
    pv_part = jnp.dot(z2h, wv[...], preferred_element_type=jnp.float32)

    @pl.when(h == 0)
    def _():
        pu_out[...] = pu_part
        pv_out[...] = pv_part

    @pl.when(h != 0)
    def _():
        pu_out[...] += pu_part
        pv_out[...] += pv_part


def _tc_layer2(sum2, cnt, z1s, W_l2, W_r2, b_l2, Wu, Wv):
    nb = NP // RB
    return pl.pallas_call(
        _tc_layer2_body,
        grid=(nb, 2),
        in_specs=[
            pl.BlockSpec((RB, HALF), lambda i, h: (i, 0)),
            pl.BlockSpec((RB, HALF), lambda i, h: (i + nb, 0)),
            pl.BlockSpec((NW, RB), lambda i, h: (0, i)),
            pl.BlockSpec((RB, HALF), lambda i, h: (i, 0)),
            pl.BlockSpec((RB, HALF), lambda i, h: (i + nb, 0)),
            pl.BlockSpec((H, HALF), lambda i, h: (0, h)),
            pl.BlockSpec((H, HALF), lambda i, h: (0, h)),
            pl.BlockSpec((1, HALF), lambda i, h: (0, h)),
            pl.BlockSpec((HALF, MH), lambda i, h: (h, 0)),
            pl.BlockSpec((HALF, MH), lambda i, h: (h, 0)),
        ],
        out_specs=[
            pl.BlockSpec((RB, MH), lambda i, h: (i, 0)),
            pl.BlockSpec((RB, MH), lambda i, h: (i, 0)),
        ],
        out_shape=[
            jax.ShapeDtypeStruct((NP, MH), jnp.float32),
            jax.ShapeDtypeStruct((NP, MH), jnp.float32),
        ],
        name="tc_layer2",
    )(sum2, sum2, cnt, z1s, z1s, W_l2, W_r2, b_l2.reshape(1, H), Wu, Wv)


def _tc_head_body(guv, ea, we, bm1, wm2, bm2, out):
    pre = (guv[...] + jnp.dot(ea[...], we[...],
                              preferred_element_type=jnp.float32) + bm1[...])
    hr = jnp.maximum(pre, 0.0)
    logit = jnp.dot(hr, wm2[...], preferred_element_type=jnp.float32) + bm2[...]
    out[...] = 1.0 / (1.0 + jnp.exp(-logit))


def _tc_head(guv, edge_attr, We, b_m1, W_m2, b_m2):
    return pl.pallas_call(
        _tc_head_body,
        grid=(E // EB,),
        in_specs=[
            pl.BlockSpec((EB, MH), lambda i: (i, 0)),
            pl.BlockSpec((EB, DE), lambda i: (i, 0)),
            pl.BlockSpec((DE, MH), lambda i: (0, 0)),
            pl.BlockSpec((1, MH), lambda i: (0, 0)),
            pl.BlockSpec((MH, 1), lambda i: (0, 0)),
            pl.BlockSpec((1, 1), lambda i: (0, 0)),
        ],
        out_specs=pl.BlockSpec((EB, 1), lambda i: (i, 0)),
        out_shape=jax.ShapeDtypeStruct((E, 1), jnp.float32),
        name="tc_head",
    )(guv, edge_attr, We, b_m1, W_m2, b_m2.reshape(1, 1))


def _pad_mh(w):
    return jnp.pad(w, ((0, 0), (0, MH - MLP_H)))


def kernel(x, edge_index, edge_u, edge_v, edge_attr,
           W_l1, b_l1, W_r1, W_l2, b_l2, W_r2,
           W_m1, b_m1, W_m2, b_m2):
    src = edge_index[0].astype(jnp.int32)
    dst = edge_index[1].astype(jnp.int32)

    src_idx = jnp.concatenate([src, src + NP])
    dst_idx = dst
    z128 = jnp.zeros((ROWS_PT, HALF), jnp.float32)

    xp = jnp.pad(x, ((0, NP - N), (0, 0)))
    x2 = jnp.concatenate([xp[:, :HALF], xp[:, HALF:]], axis=0)
    sum1, cnt = _seg_sum_call(x2, src_idx, dst_idx, z128, True)
    cnt = cnt.reshape(NW, NP)
    z1s = _tc_layer1(sum1, cnt, xp, W_l1, W_r1, b_l1)

    sum2 = _seg_sum_call(z1s, src_idx, dst_idx, z128, False)
    Wu = _pad_mh(W_m1[:H])
    Wv = _pad_mh(W_m1[H:2 * H])
    We = _pad_mh(W_m1[2 * H:])
    bm1 = _pad_mh(b_m1.reshape(1, MLP_H))
    wm2 = jnp.pad(W_m2, ((0, MH - MLP_H), (0, 0)))
    pu, pv = _tc_layer2(sum2, cnt, z1s, W_l2, W_r2, b_l2, Wu, Wv)

    pq = jnp.concatenate([pu, pv], axis=0)
    pad_c = NCH_C * CHUNK_C - EPT_C
    u_r = jnp.pad(edge_u.astype(jnp.int32).reshape(NW, EPT_C),
                  ((0, 0), (0, pad_c))).reshape(NW, NCH_C, CHUNK_C)
    v_r = jnp.pad(edge_v.astype(jnp.int32).reshape(NW, EPT_C),
                  ((0, 0), (0, pad_c))).reshape(NW, NCH_C, CHUNK_C) + NP
    uv_idx = jnp.concatenate([u_r, v_r], axis=0)
    guv = _edge_gather(pq, uv_idx)
    out = _tc_head(guv, edge_attr, We, bm1, wm2, b_m2)
    return out.reshape(E)


# speedup vs baseline: 1.2910x; 1.0272x over previous
import functools

import jax
import jax.numpy as jnp
from jax import lax
from jax.experimental import pallas as pl
from jax.experimental.pallas import tpu as pltpu
from jax.experimental.pallas import tpu_sc as plsc

N = 10000
E = 160000
D = 256
H = 256
DE = 16
MLP_H = 96

NP = 10240
MH = 128
NC = 2
NS = 16
NW = NC * NS
HALF = D // 2
L = 16

EPT_A = E // NS
CHUNK_A = 80
NCH_A = EPT_A // CHUNK_A
ROWS_PT = NP // NS

EPT_C = E // NW
CHUNK_C = 128
NFULL_C = EPT_C // CHUNK_C
TAIL_C = EPT_C - NFULL_C * CHUNK_C
NCH_C = NFULL_C + 1

RB = 2048
EB = 8000

_SC_MESH = plsc.VectorSubcoreMesh(core_axis_name="c", subcore_axis_name="s")
_SC_PARAMS = pltpu.CompilerParams(needs_layout_passes=False)


def _seg_sum_body(with_cnt, x2, src_off, dst, z128,
                  *refs):
    if with_cnt:
        (sum_out, cnt_out, is0, id0, is1, id1, rb0, rb1, hist, acc_sp,
         sg0, sg1, si0, si1) = refs
    else:
        (sum_out, is0, id0, is1, id1, rb0, rb1, acc_sp,
         sg0, sg1, si0, si1) = refs
    c = lax.axis_index("c")
    s = lax.axis_index("s")
    row0 = s * ROWS_PT
    pltpu.sync_copy(z128, acc_sp.at[pl.ds(row0, ROWS_PT)])
    if with_cnt:
        zv = jnp.zeros((L,), jnp.float32)

        def zrow(k, cc):
            hist[pl.ds(k * L, L)] = zv
            return cc

        lax.fori_loop(0, NP // L, zrow, 0)
    plsc.subcore_barrier()
    base_s = c * E + s * EPT_A
    base_d = s * EPT_A
    ones_v = jnp.full((L,), 1.0, jnp.float32)

    def issue_idx(j, is_b, id_b, si):
        pltpu.async_copy(src_off.at[pl.ds(base_s + j * CHUNK_A, CHUNK_A)],
                         is_b, si)
        pltpu.async_copy(dst.at[pl.ds(base_d + j * CHUNK_A, CHUNK_A)],
                         id_b, si)

    def wait_idx(is_b, id_b, si):
        pltpu.make_async_copy(src_off.at[pl.ds(0, CHUNK_A)], is_b, si).wait()
        pltpu.make_async_copy(dst.at[pl.ds(0, CHUNK_A)], id_b, si).wait()

    def issue_gather(is_b, rb, sg):
        pltpu.async_copy(x2.at[is_b], rb, sg)

    def wait_gather(rb, sg):
        pltpu.make_async_copy(x2.at[pl.ds(0, CHUNK_A)], rb, sg).wait()

    def consume(rb, id_b):
        pltpu.sync_copy(rb, acc_sp.at[id_b], add=True)
        if with_cnt:
            for t in range(CHUNK_A // L):
                idx16 = id_b[pl.ds(t * L, L)]
                plsc.addupdate_scatter(hist, [idx16], ones_v)

    issue_idx(0, is0, id0, si0)
    issue_idx(1, is1, id1, si1)
    wait_idx(is0, id0, si0)
    issue_gather(is0, rb0, sg0)
    npair = (NCH_A - 1) // 2

    def pipe(g, carry):
        j0 = 2 * g
        wait_idx(is1, id1, si1)
        issue_gather(is1, rb1, sg1)
        wait_gather(rb0, sg0)
        consume(rb0, id0)
        issue_idx(j0 + 2, is0, id0, si0)
        wait_gather(rb1, sg1)
        consume(rb1, id1)

        @pl.when(g < npair - 1)
        def _():
            issue_idx(j0 + 3, is1, id1, si1)

        wait_idx(is0, id0, si0)
        issue_gather(is0, rb0, sg0)
        return carry

    lax.fori_loop(0, npair, pipe, 0)
    wait_gather(rb0, sg0)
    consume(rb0, id0)
    plsc.subcore_barrier()
    pltpu.sync_copy(acc_sp.at[pl.ds(row0, ROWS_PT)],
                    sum_out.at[pl.ds(c * NP + row0, ROWS_PT)])
    if with_cnt:
        w = c * NS + s
        pltpu.sync_copy(hist, cnt_out.at[pl.ds(w * NP, NP)])


def _seg_sum_call(x2, src_idx, dst_idx, z128, with_cnt):
    out_type = [jax.ShapeDtypeStruct((2 * NP, HALF), jnp.float32)]
    scratch = [
        pltpu.VMEM((CHUNK_A,), jnp.int32),
        pltpu.VMEM((CHUNK_A,), jnp.int32),
        pltpu.VMEM((CHUNK_A,), jnp.int32),
        pltpu.VMEM((CHUNK_A,), jnp.int32),
        pltpu.VMEM((CHUNK_A, HALF), jnp.float32),
        pltpu.VMEM((CHUNK_A, HALF), jnp.float32),
    ]
    if with_cnt:
        out_type.append(jax.ShapeDtypeStruct((NW * NP,), jnp.float32))
        scratch.append(pltpu.VMEM((NP,), jnp.float32))
    scratch.append(pltpu.VMEM_SHARED((NP, HALF), jnp.float32))
    scratch.extend([pltpu.SemaphoreType.DMA] * 4)
    fn = pl.kernel(
        functools.partial(_seg_sum_body, with_cnt),
        out_type=tuple(out_type) if with_cnt else out_type[0],
        mesh=_SC_MESH,
        scratch_types=tuple(scratch),
        compiler_params=_SC_PARAMS,
        name="sc_seg_sum" + ("_cnt" if with_cnt else ""),
    )
    return fn(x2, src_idx, dst_idx, z128)


def _edge_gather_body(pu_t, pv_t, u_idx, v_idx, out,
                      idx_u, idx_v, ru0, rv0, ru1, rv1, ru2, rv2,
                      sg0, sg1, sg2, ss0, ss1, ss2):
    c = lax.axis_index("c")
    s = lax.axis_index("s")
    w = c * NS + s
    pltpu.sync_copy(u_idx.at[w], idx_u)
    pltpu.sync_copy(v_idx.at[w], idx_v)

    def issue(j, ru_b, rv_b, sg):
        pltpu.async_copy(pu_t.at[idx_u.at[j]], ru_b, sg)
        pltpu.async_copy(pv_t.at[idx_v.at[j]], rv_b, sg)

    def wait2(ru_b, rv_b, sg):
        pltpu.make_async_copy(pu_t.at[pl.ds(0, CHUNK_C)], ru_b, sg).wait()
        pltpu.make_async_copy(pu_t.at[pl.ds(0, CHUNK_C)], rv_b, sg).wait()

    def addrows(ru_b, rv_b, n):
        def addrow(k, cc):
            for r in range(2):
                for t in range(MH // L):
                    sl = pl.ds(t * L, L)
                    ru_b[2 * k + r, sl] = ru_b[2 * k + r, sl] + rv_b[2 * k + r, sl]
            return cc

        lax.fori_loop(0, n // 2, addrow, 0)

    def store_async(j, ru_b, ss):
        pltpu.async_copy(
            ru_b, out.at[pl.ds(w * EPT_C + j * CHUNK_C, CHUNK_C)], ss)

    def wait_store(ru_b, ss):
        pltpu.make_async_copy(pu_t.at[pl.ds(0, CHUNK_C)], ru_b, ss).wait()

    def consume(j, ru_b, rv_b, sg, ss):
        wait2(ru_b, rv_b, sg)
        addrows(ru_b, rv_b, CHUNK_C)
        store_async(j, ru_b, ss)

    issue(0, ru0, rv0, sg0)
    issue(1, ru1, rv1, sg1)
    consume(0, ru0, rv0, sg0, ss0)
    issue(2, ru2, rv2, sg2)
    consume(1, ru1, rv1, sg1, ss1)
    wait_store(ru0, ss0)
    issue(3, ru0, rv0, sg0)

    def pipe(g, carry):
        j0 = 3 * g
        consume(j0 - 1, ru2, rv2, sg2, ss2)
        wait_store(ru1, ss1)
        issue(j0 + 1, ru1, rv1, sg1)
        consume(j0, ru0, rv0, sg0, ss0)
        wait_store(ru2, ss2)
        issue(j0 + 2, ru2, rv2, sg2)
        consume(j0 + 1, ru1, rv1, sg1, ss1)
        wait_store(ru0, ss0)
        issue(j0 + 3, ru0, rv0, sg0)
        return carry

    lax.fori_loop(1, 13, pipe, 0)
    consume(NFULL_C - 1, ru2, rv2, sg2, ss2)
    wait2(ru0, rv0, sg0)
    addrows(ru0, rv0, TAIL_C)
    wait_store(ru1, ss1)
    wait_store(ru2, ss2)
    pltpu.sync_copy(ru0.at[pl.ds(0, TAIL_C)],
                    out.at[pl.ds(w * EPT_C + NFULL_C * CHUNK_C, TAIL_C)])


_edge_gather = pl.kernel(
    _edge_gather_body,
    out_type=jax.ShapeDtypeStruct((E, MH), jnp.float32),
    mesh=_SC_MESH,
    scratch_types=(
        pltpu.VMEM((NCH_C, CHUNK_C), jnp.int32),
        pltpu.VMEM((NCH_C, CHUNK_C), jnp.int32),
        pltpu.VMEM((CHUNK_C, MH), jnp.float32),
        pltpu.VMEM((CHUNK_C, MH), jnp.float32),
        pltpu.VMEM((CHUNK_C, MH), jnp.float32),
        pltpu.VMEM((CHUNK_C, MH), jnp.float32),
        pltpu.VMEM((CHUNK_C, MH), jnp.float32),
        pltpu.VMEM((CHUNK_C, MH), jnp.float32),
        pltpu.SemaphoreType.DMA,
        pltpu.SemaphoreType.DMA,
        pltpu.SemaphoreType.DMA,
        pltpu.SemaphoreType.DMA,
        pltpu.SemaphoreType.DMA,
        pltpu.SemaphoreType.DMA,
    ),
    compiler_params=_SC_PARAMS,
    name="sc_edge_gather",
)


def _inv_degree(cnt_block):
    deg2 = jnp.sum(jnp.transpose(cnt_block), axis=1, keepdims=True)
    return 1.0 / jnp.maximum(0.5 * deg2, 1.0)


def _tc_layer1_body(sumA, sumB, cnt, x, wl, wr, b, out):
    inv = _inv_degree(cnt[...])
    mean = jnp.concatenate([sumA[...], sumB[...]], axis=1) * inv
    acc = jnp.dot(mean, wl[...], preferred_element_type=jnp.float32)
    acc += jnp.dot(x[...], wr[...], preferred_element_type=jnp.float32)
    out[...] = jnp.maximum(acc + b[...], 0.0)


def _tc_layer1(sum1, cnt, xp, W_l1, W_r1, b_l1):
    nb = NP // RB
    return pl.pallas_call(
        _tc_layer1_body,
        grid=(nb, 2),
        in_specs=[
            pl.BlockSpec((RB, HALF), lambda i, h: (i, 0)),
            pl.BlockSpec((RB, HALF), lambda i, h: (i + nb, 0)),
            pl.BlockSpec((NW, RB), lambda i, h: (0, i)),
            pl.BlockSpec((RB, D), lambda i, h: (i, 0)),
            pl.BlockSpec((D, HALF), lambda i, h: (0, h)),
            pl.BlockSpec((D, HALF), lambda i, h: (0, h)),
            pl.BlockSpec((1, HALF), lambda i, h: (0, h)),
        ],
        out_specs=pl.BlockSpec((RB, HALF), lambda i, h: (h * nb + i, 0)),
        out_shape=jax.ShapeDtypeStruct((2 * NP, HALF), jnp.float32),
        name="tc_layer1",
    )(sum1, sum1, cnt, xp, W_l1, W_r1, b_l1.reshape(1, H))


def _tc_layer2_body(sumA, sumB, cnt, z1A, z1B, wl, wr, b, wu, wv,
                    pu_out, pv_out):
    h = pl.program_id(1)
    inv = _inv_degree(cnt[...])
    mean = jnp.concatenate([sumA[...], sumB[...]], axis=1) * inv
    z1 = jnp.concatenate([z1A[...], z1B[...]], axis=1)
    z2h = (jnp.dot(mean, wl[...], preferred_element_type=jnp.float32)
           + jnp.dot(z1, wr[...], preferred_element_type=jnp.float32)
           + b[...])
    pu_part = jnp.dot(z2h, wu[...], preferred_element_type=jnp.float32)
    pv_part = jnp.dot(z2h, wv[...], preferred_element_type=jnp.float32)

    @pl.when(h == 0)
    def _():
        pu_out[...] = pu_part
        pv_out[...] = pv_part

    @pl.when(h != 0)
    def _():
        pu_out[...] += pu_part
        pv_out[...] += pv_part


def _tc_layer2(sum2, cnt, z1s, W_l2, W_r2, b_l2, Wu, Wv):
    nb = NP // RB
    return pl.pallas_call(
        _tc_layer2_body,
        grid=(nb, 2),
        in_specs=[
            pl.BlockSpec((RB, HALF), lambda i, h: (i, 0)),
            pl.BlockSpec((RB, HALF), lambda i, h: (i + nb, 0)),
            pl.BlockSpec((NW, RB), lambda i, h: (0, i)),
            pl.BlockSpec((RB, HALF), lambda i, h: (i, 0)),
            pl.BlockSpec((RB, HALF), lambda i, h: (i + nb, 0)),
            pl.BlockSpec((H, HALF), lambda i, h: (0, h)),
            pl.BlockSpec((H, HALF), lambda i, h: (0, h)),
            pl.BlockSpec((1, HALF), lambda i, h: (0, h)),
            pl.BlockSpec((HALF, MH), lambda i, h: (h, 0)),
            pl.BlockSpec((HALF, MH), lambda i, h: (h, 0)),
        ],
        out_specs=[
            pl.BlockSpec((RB, MH), lambda i, h: (i, 0)),
            pl.BlockSpec((RB, MH), lambda i, h: (i, 0)),
        ],
        out_shape=[
            jax.ShapeDtypeStruct((NP, MH), jnp.float32),
            jax.ShapeDtypeStruct((NP, MH), jnp.float32),
        ],
        name="tc_layer2",
    )(sum2, sum2, cnt, z1s, z1s, W_l2, W_r2, b_l2.reshape(1, H), Wu, Wv)


def _tc_head_body(guv, ea, we, bm1, wm2, bm2, out):
    pre = (guv[...] + jnp.dot(ea[...], we[...],
                              preferred_element_type=jnp.float32) + bm1[...])
    hr = jnp.maximum(pre, 0.0)
    logit = jnp.dot(hr, wm2[...], preferred_element_type=jnp.float32) + bm2[...]
    out[...] = 1.0 / (1.0 + jnp.exp(-logit))


def _tc_head(guv, edge_attr, We, b_m1, W_m2, b_m2):
    return pl.pallas_call(
        _tc_head_body,
        grid=(E // EB,),
        in_specs=[
            pl.BlockSpec((EB, MH), lambda i: (i, 0)),
            pl.BlockSpec((EB, DE), lambda i: (i, 0)),
            pl.BlockSpec((DE, MH), lambda i: (0, 0)),
            pl.BlockSpec((1, MH), lambda i: (0, 0)),
            pl.BlockSpec((MH, 1), lambda i: (0, 0)),
            pl.BlockSpec((1, 1), lambda i: (0, 0)),
        ],
        out_specs=pl.BlockSpec((EB, 1), lambda i: (i, 0)),
        out_shape=jax.ShapeDtypeStruct((E, 1), jnp.float32),
        name="tc_head",
    )(guv, edge_attr, We, b_m1, W_m2, b_m2.reshape(1, 1))


def _pad_mh(w):
    return jnp.pad(w, ((0, 0), (0, MH - MLP_H)))


def kernel(x, edge_index, edge_u, edge_v, edge_attr,
           W_l1, b_l1, W_r1, W_l2, b_l2, W_r2,
           W_m1, b_m1, W_m2, b_m2):
    src = edge_index[0].astype(jnp.int32)
    dst = edge_index[1].astype(jnp.int32)

    src_idx = jnp.concatenate([src, src + NP])
    dst_idx = dst
    z128 = jnp.zeros((ROWS_PT, HALF), jnp.float32)

    xp = jnp.pad(x, ((0, NP - N), (0, 0)))
    x2 = jnp.concatenate([xp[:, :HALF], xp[:, HALF:]], axis=0)
    sum1, cnt = _seg_sum_call(x2, src_idx, dst_idx, z128, True)
    cnt = cnt.reshape(NW, NP)
    z1s = _tc_layer1(sum1, cnt, xp, W_l1, W_r1, b_l1)

    sum2 = _seg_sum_call(z1s, src_idx, dst_idx, z128, False)
    Wu = _pad_mh(W_m1[:H])
    Wv = _pad_mh(W_m1[H:2 * H])
    We = _pad_mh(W_m1[2 * H:])
    bm1 = _pad_mh(b_m1.reshape(1, MLP_H))
    wm2 = jnp.pad(W_m2, ((0, MH - MLP_H), (0, 0)))
    pu, pv = _tc_layer2(sum2, cnt, z1s, W_l2, W_r2, b_l2, Wu, Wv)

    pad_c = NCH_C * CHUNK_C - EPT_C
    u_r = jnp.pad(edge_u.astype(jnp.int32).reshape(NW, EPT_C),
                  ((0, 0), (0, pad_c))).reshape(NW, NCH_C, CHUNK_C)
    v_r = jnp.pad(edge_v.astype(jnp.int32).reshape(NW, EPT_C),
                  ((0, 0), (0, pad_c))).reshape(NW, NCH_C, CHUNK_C)
    guv = _edge_gather(pu, pv, u_r, v_r)
    out = _tc_head(guv, edge_attr, We, bm1, wm2, b_m2)
    return out.reshape(E)
